# baseline (speedup 1.0000x reference)
import functools

import jax
import jax.numpy as jnp
from jax import lax
from jax.experimental import pallas as pl
from jax.experimental.pallas import tpu as pltpu
from jax.experimental.pallas import tpu_sc as plsc

N_FACTORS = 16
BATCH = 16384
NUM_WORKERS = 32
B_PER_W = BATCH // NUM_WORKERS
GCHUNK = 128
N_GCHUNKS = B_PER_W // GCHUNK
CCHUNK = 16


def _body(u_hbm, i_hbm, user_hbm, item_hbm, out_hbm,
          idx_u, idx_i, rows_u, rows_i, out_v, sem):
    wid = lax.axis_index("s") * 2 + lax.axis_index("c")
    base = pl.multiple_of(wid * B_PER_W, B_PER_W)

    pltpu.sync_copy(u_hbm.at[pl.ds(base, B_PER_W)], idx_u)
    pltpu.sync_copy(i_hbm.at[pl.ds(base, B_PER_W)], idx_i)

    copies = []
    for j in range(N_GCHUNKS):
        sl = pl.ds(j * GCHUNK, GCHUNK)
        copies.append(pltpu.async_copy(
            user_hbm.at[idx_u.at[sl]], rows_u.at[sl], sem))
        copies.append(pltpu.async_copy(
            item_hbm.at[idx_i.at[sl]], rows_i.at[sl], sem))
    for c in copies:
        c.wait()

    lanes = lax.iota(jnp.int32, CCHUNK)

    def step(cidx, _):
        b0 = cidx * CCHUNK
        svec = jnp.zeros((CCHUNK,), jnp.float32)
        for k in range(CCHUNK):
            prod = rows_u[b0 + k] * rows_i[b0 + k]
            s = jnp.sum(prod)
            svec = jnp.where(lanes == k, s, svec)
        out_v[pl.ds(pl.multiple_of(b0, CCHUNK), CCHUNK)] = svec
        return _

    lax.fori_loop(0, B_PER_W // CCHUNK, step, None)

    pltpu.sync_copy(out_v, out_hbm.at[pl.ds(base, B_PER_W)])


@jax.jit
def kernel(u, i, user_emb, item_emb):
    mesh = plsc.VectorSubcoreMesh(core_axis_name="c", subcore_axis_name="s")
    run = pl.kernel(
        _body,
        mesh=mesh,
        out_type=jax.ShapeDtypeStruct((BATCH,), jnp.float32),
        scratch_types=[
            pltpu.VMEM((B_PER_W,), jnp.int32),
            pltpu.VMEM((B_PER_W,), jnp.int32),
            pltpu.VMEM((B_PER_W, N_FACTORS), jnp.float32),
            pltpu.VMEM((B_PER_W, N_FACTORS), jnp.float32),
            pltpu.VMEM((B_PER_W,), jnp.float32),
            pltpu.SemaphoreType.DMA,
        ],
        compiler_params=pltpu.CompilerParams(
            needs_layout_passes=False, use_tc_tiling_on_sc=False),
    )
    return run(u, i, user_emb, item_emb)
